# baseline (speedup 1.0000x reference)
import functools

import jax
import jax.numpy as jnp
from jax import lax
from jax.experimental import pallas as pl
from jax.experimental.pallas import tpu as pltpu
from jax.experimental.pallas import tpu_sc as plsc

_N_TOK = 8192
_D = 2048
_SQRT_K = 8
_NUM_EXPERTS = 64
_TOP_K = 8
_GATE_RANK = 16

_BLK = 1024
_NC = 2
_NS = 16
_NW = _NC * _NS
_CHUNKS = 1
_CTOK = _N_TOK // _CHUNKS
_TPW = _CTOK // _NW


def _dense_body(x_ref, wct_ref, gt_ref, sel_ref, tot_ref):
    x = x_ref[...].astype(jnp.bfloat16)
    y = jnp.dot(x, wct_ref[...].astype(jnp.bfloat16),
                preferred_element_type=jnp.float32)
    s1 = y[:, 0:_SQRT_K]
    s2 = y[:, _SQRT_K:2 * _SQRT_K]
    qg = y[:, 2 * _SQRT_K:]

    col = lax.broadcasted_iota(jnp.int32, (_SQRT_K, _NUM_EXPERTS), 1)
    row = lax.broadcasted_iota(jnp.int32, (_SQRT_K, _NUM_EXPERTS), 0)
    a = ((col // _SQRT_K) == row).astype(jnp.float32)
    b = ((col % _SQRT_K) == row).astype(jnp.float32)
    hi = lax.Precision.HIGHEST
    sel = (jnp.dot(s1, a, precision=hi, preferred_element_type=jnp.float32)
           + jnp.dot(s2, b, precision=hi, preferred_element_type=jnp.float32))
    gate = jnp.dot(qg.astype(jnp.bfloat16), gt_ref[...].astype(jnp.bfloat16),
                   preferred_element_type=jnp.float32)
    sel_ref[...] = sel
    tot_ref[...] = sel + gate


_sc_mesh = plsc.VectorSubcoreMesh(core_axis_name="c", subcore_axis_name="s")


@functools.partial(
    pl.kernel,
    mesh=_sc_mesh,
    out_type=[
        jax.ShapeDtypeStruct((_CTOK * _TOP_K,), jnp.int32),
        jax.ShapeDtypeStruct((_CTOK * _TOP_K,), jnp.float32),
    ],
    scratch_types=[
        pltpu.VMEM((_TPW * _NUM_EXPERTS,), jnp.float32),
        pltpu.VMEM((_TPW * _NUM_EXPERTS,), jnp.float32),
        pltpu.VMEM((_TPW * _TOP_K + 8,), jnp.int32),
        pltpu.VMEM((_TPW * _TOP_K + 8,), jnp.float32),
        pltpu.SemaphoreType.DMA,
    ],
    compiler_params=pltpu.CompilerParams(needs_layout_passes=False),
)
def _sc_router(sel_hbm, tot_hbm, idx_hbm, wts_hbm,
               sel_v, tot_v, idx_v, wts_v, sem):
    wid = lax.axis_index("s") * _NC + lax.axis_index("c")
    in_base = wid * _TPW * _NUM_EXPERTS
    out_base = wid * _TPW * _TOP_K
    pltpu.async_copy(sel_hbm.at[pl.ds(in_base, _TPW * _NUM_EXPERTS)],
                     sel_v, sem).wait()
    pltpu.async_copy(tot_hbm.at[pl.ds(in_base, _TPW * _NUM_EXPERTS)],
                     tot_v, sem).wait()

    lane = lax.iota(jnp.int32, 16)
    low8 = lane < 8

    def merge(pk, pv, qk, qv):
        rk = lax.rev(qk, (0,))
        rv = lax.rev(qv, (0,))
        take = pk >= rk
        return plsc.sort_key_val(jnp.where(take, pk, rk),
                                 jnp.where(take, pv, rv),
                                 descending=True)

    def one_token(t):
        base = t * _NUM_EXPERTS
        parts = []
        for c in range(4):
            k = sel_v[pl.ds(base + 16 * c, 16)]
            i = lane + 16 * c
            parts.append(plsc.sort_key_val(k, i, descending=True))
        m01 = merge(parts[0][0], parts[0][1], parts[1][0], parts[1][1])
        m23 = merge(parts[2][0], parts[2][1], parts[3][0], parts[3][1])
        fk, fi = merge(m01[0], m01[1], m23[0], m23[1])
        tv = plsc.load_gather(tot_v, [base + fi])
        mx = jnp.max(jnp.where(low8, tv, -3.4e38), axis=0)
        e = jnp.where(low8, jnp.exp(tv - mx), 0.0)
        w = e / jnp.sum(e, axis=0)
        plsc.store_compressed(idx_v.at[pl.ds(t * _TOP_K, 16)], fi, mask=low8)
        plsc.store_compressed(wts_v.at[pl.ds(t * _TOP_K, 16)], w, mask=low8)

    @plsc.parallel_loop(0, _TPW, step=1, unroll=4)
    def _loop(t):
        one_token(t)

    pltpu.async_copy(idx_v.at[pl.ds(0, _TPW * _TOP_K)],
                     idx_hbm.at[pl.ds(out_base, _TPW * _TOP_K)], sem).wait()
    pltpu.async_copy(wts_v.at[pl.ds(0, _TPW * _TOP_K)],
                     wts_hbm.at[pl.ds(out_base, _TPW * _TOP_K)], sem).wait()


def _dense_chunk(xc, wct, gt):
    grid = (_CTOK // _BLK,)
    return pl.pallas_call(
        _dense_body,
        grid=grid,
        in_specs=[
            pl.BlockSpec((_BLK, _D), lambda i: (i, 0)),
            pl.BlockSpec((_D, 2 * _SQRT_K + _GATE_RANK), lambda i: (0, 0)),
            pl.BlockSpec((_GATE_RANK, _NUM_EXPERTS), lambda i: (0, 0)),
        ],
        out_specs=[
            pl.BlockSpec((_BLK, _NUM_EXPERTS), lambda i: (i, 0)),
            pl.BlockSpec((_BLK, _NUM_EXPERTS), lambda i: (i, 0)),
        ],
        out_shape=[
            jax.ShapeDtypeStruct((_CTOK, _NUM_EXPERTS), jnp.float32),
            jax.ShapeDtypeStruct((_CTOK, _NUM_EXPERTS), jnp.float32),
        ],
        compiler_params=pltpu.CompilerParams(
            dimension_semantics=("arbitrary",),
        ),
    )(xc, wct, gt)


@jax.jit
def kernel(x, W1, W2, Wg, G):
    wct = jnp.concatenate([W1, W2, Wg], axis=0).T
    gt = G.T
    sels, idxs, wtss = [], [], []
    for ch in range(_CHUNKS):
        xc = lax.slice_in_dim(x, ch * _CTOK, (ch + 1) * _CTOK, axis=0)
        sel, tot = _dense_chunk(xc, wct, gt)
        idx_flat, wts_flat = _sc_router(sel.reshape(-1), tot.reshape(-1))
        sels.append(sel)
        idxs.append(idx_flat.reshape(_CTOK, _TOP_K))
        wtss.append(wts_flat.reshape(_CTOK, _TOP_K))
    if _CHUNKS == 1:
        return idxs[0], wtss[0], sels[0]
    return (jnp.concatenate(idxs, axis=0),
            jnp.concatenate(wtss, axis=0),
            jnp.concatenate(sels, axis=0))


# speedup vs baseline: 1.0114x; 1.0114x over previous
import functools

import jax
import jax.numpy as jnp
from jax import lax
from jax.experimental import pallas as pl
from jax.experimental.pallas import tpu as pltpu
from jax.experimental.pallas import tpu_sc as plsc

_N_TOK = 8192
_D = 2048
_SQRT_K = 8
_NUM_EXPERTS = 64
_TOP_K = 8
_GATE_RANK = 16

_BLK = 1024
_NC = 2
_NS = 16
_NW = _NC * _NS
_CHUNKS = 1
_CTOK = _N_TOK // _CHUNKS
_TPW = _CTOK // _NW


def _dense_body(x_ref, wct_ref, gt_ref, sel_ref, tot_ref):
    x = x_ref[...].astype(jnp.bfloat16)
    y = jnp.dot(x, wct_ref[...].astype(jnp.bfloat16),
                preferred_element_type=jnp.float32)
    s1 = y[:, 0:_SQRT_K]
    s2 = y[:, _SQRT_K:2 * _SQRT_K]
    qg = y[:, 2 * _SQRT_K:]

    col = lax.broadcasted_iota(jnp.int32, (_SQRT_K, _NUM_EXPERTS), 1)
    row = lax.broadcasted_iota(jnp.int32, (_SQRT_K, _NUM_EXPERTS), 0)
    a = ((col // _SQRT_K) == row).astype(jnp.float32)
    b = ((col % _SQRT_K) == row).astype(jnp.float32)
    hi = lax.Precision.HIGHEST
    sel = (jnp.dot(s1, a, precision=hi, preferred_element_type=jnp.float32)
           + jnp.dot(s2, b, precision=hi, preferred_element_type=jnp.float32))
    gate = jnp.dot(qg.astype(jnp.bfloat16), gt_ref[...].astype(jnp.bfloat16),
                   preferred_element_type=jnp.float32)
    sel_ref[...] = sel
    tot_ref[...] = sel + gate


_sc_mesh = plsc.VectorSubcoreMesh(core_axis_name="c", subcore_axis_name="s")


@functools.partial(
    pl.kernel,
    mesh=_sc_mesh,
    out_type=[
        jax.ShapeDtypeStruct((_CTOK * _TOP_K,), jnp.int32),
        jax.ShapeDtypeStruct((_CTOK * _TOP_K,), jnp.float32),
    ],
    scratch_types=[
        pltpu.VMEM((_TPW * _NUM_EXPERTS,), jnp.float32),
        pltpu.VMEM((_TPW * _NUM_EXPERTS,), jnp.float32),
        pltpu.VMEM((_TPW * _TOP_K + 8,), jnp.int32),
        pltpu.VMEM((_TPW * _TOP_K + 8,), jnp.float32),
        pltpu.SemaphoreType.DMA,
    ],
    compiler_params=pltpu.CompilerParams(needs_layout_passes=False),
)
def _sc_router(sel_hbm, tot_hbm, idx_hbm, wts_hbm,
               sel_v, tot_v, idx_v, wts_v, sem):
    wid = lax.axis_index("s") * _NC + lax.axis_index("c")
    in_base = wid * _TPW * _NUM_EXPERTS
    out_base = wid * _TPW * _TOP_K
    cp1 = pltpu.async_copy(sel_hbm.at[pl.ds(in_base, _TPW * _NUM_EXPERTS)],
                           sel_v, sem)
    cp2 = pltpu.async_copy(tot_hbm.at[pl.ds(in_base, _TPW * _NUM_EXPERTS)],
                           tot_v, sem)
    cp1.wait()
    cp2.wait()

    lane = lax.iota(jnp.int32, 16)
    low8 = lane < 8

    def merge(pk, pv, qk, qv):
        rk = lax.rev(qk, (0,))
        rv = lax.rev(qv, (0,))
        take = pk >= rk
        return plsc.sort_key_val(jnp.where(take, pk, rk),
                                 jnp.where(take, pv, rv),
                                 descending=True)

    def one_token(t):
        base = t * _NUM_EXPERTS
        parts = []
        for c in range(4):
            k = sel_v[pl.ds(base + 16 * c, 16)]
            i = lane + 16 * c
            parts.append(plsc.sort_key_val(k, i, descending=True))
        m01 = merge(parts[0][0], parts[0][1], parts[1][0], parts[1][1])
        m23 = merge(parts[2][0], parts[2][1], parts[3][0], parts[3][1])
        fk, fi = merge(m01[0], m01[1], m23[0], m23[1])
        tv = plsc.load_gather(tot_v, [base + fi])
        mx = jnp.max(jnp.where(low8, tv, -3.4e38), axis=0)
        e = jnp.where(low8, jnp.exp(tv - mx), 0.0)
        w = e / jnp.sum(e, axis=0)
        plsc.store_compressed(idx_v.at[pl.ds(t * _TOP_K, 16)], fi, mask=low8)
        plsc.store_compressed(wts_v.at[pl.ds(t * _TOP_K, 16)], w, mask=low8)

    @plsc.parallel_loop(0, _TPW, step=1, unroll=4)
    def _loop(t):
        one_token(t)

    pltpu.async_copy(idx_v.at[pl.ds(0, _TPW * _TOP_K)],
                     idx_hbm.at[pl.ds(out_base, _TPW * _TOP_K)], sem).wait()
    pltpu.async_copy(wts_v.at[pl.ds(0, _TPW * _TOP_K)],
                     wts_hbm.at[pl.ds(out_base, _TPW * _TOP_K)], sem).wait()


def _dense_chunk(xc, wct, gt):
    grid = (_CTOK // _BLK,)
    return pl.pallas_call(
        _dense_body,
        grid=grid,
        in_specs=[
            pl.BlockSpec((_BLK, _D), lambda i: (i, 0)),
            pl.BlockSpec((_D, 2 * _SQRT_K + _GATE_RANK), lambda i: (0, 0)),
            pl.BlockSpec((_GATE_RANK, _NUM_EXPERTS), lambda i: (0, 0)),
        ],
        out_specs=[
            pl.BlockSpec((_BLK, _NUM_EXPERTS), lambda i: (i, 0)),
            pl.BlockSpec((_BLK, _NUM_EXPERTS), lambda i: (i, 0)),
        ],
        out_shape=[
            jax.ShapeDtypeStruct((_CTOK, _NUM_EXPERTS), jnp.float32),
            jax.ShapeDtypeStruct((_CTOK, _NUM_EXPERTS), jnp.float32),
        ],
        compiler_params=pltpu.CompilerParams(
            dimension_semantics=("arbitrary",),
        ),
    )(xc, wct, gt)


@jax.jit
def kernel(x, W1, W2, Wg, G):
    wct = jnp.concatenate([W1, W2, Wg], axis=0).T
    gt = G.T
    sels, idxs, wtss = [], [], []
    for ch in range(_CHUNKS):
        xc = lax.slice_in_dim(x, ch * _CTOK, (ch + 1) * _CTOK, axis=0)
        sel, tot = _dense_chunk(xc, wct, gt)
        idx_flat, wts_flat = _sc_router(sel.reshape(-1), tot.reshape(-1))
        sels.append(sel)
        idxs.append(idx_flat.reshape(_CTOK, _TOP_K))
        wtss.append(wts_flat.reshape(_CTOK, _TOP_K))
    if _CHUNKS == 1:
        return idxs[0], wtss[0], sels[0]
    return (jnp.concatenate(idxs, axis=0),
            jnp.concatenate(wtss, axis=0),
            jnp.concatenate(sels, axis=0))
